# baseline (speedup 1.0000x reference)
import functools

import jax
import jax.numpy as jnp
from jax import lax
from jax.experimental import pallas as pl
from jax.experimental.pallas import tpu as pltpu
from jax.experimental.pallas import tpu_sc as plsc

_B = 20
_L = 20
_V = 1000
_NS = 16
_NP = _L * _L
_CHUNKS = ((0, 8), (8, 8), (16, 4))
_LN2 = 0.6931471805599453


def _clamp(x, lo, hi):
    return jnp.minimum(jnp.maximum(x, lo), hi)


def _ln16(z):
    bits = plsc.bitcast(z, jnp.int32)
    e = lax.shift_right_logical(bits, 23) & 0xFF
    ef = (e - 127).astype(jnp.float32)
    m = plsc.bitcast((bits & 0x7FFFFF) | 0x3F800000, jnp.float32)
    t = (m - 1.0) / (m + 1.0)
    t2 = t * t
    p = jnp.float32(2.0 / 13.0)
    for coef in (2.0 / 11.0, 2.0 / 9.0, 2.0 / 7.0, 2.0 / 5.0, 2.0 / 3.0, 2.0):
        p = p * t2 + jnp.float32(coef)
    return ef * jnp.float32(_LN2) + t * p


@functools.partial(
    pl.kernel,
    out_type=jax.ShapeDtypeStruct((32, 1, 16), jnp.float32),
    mesh=plsc.VectorSubcoreMesh(core_axis_name="c", subcore_axis_name="s"),
    compiler_params=pltpu.CompilerParams(needs_layout_passes=False),
    scratch_types=[
        pltpu.VMEM((_L, _V), jnp.float32),
        pltpu.VMEM((_B, _L), jnp.int32),
        pltpu.VMEM((_L * 32,), jnp.float32),
        pltpu.VMEM((16,), jnp.float32),
        pltpu.SemaphoreType.DMA,
        pltpu.SemaphoreType.DMA,
    ],
)
def _calcs_sc(tp_hbm, lbl_hbm, out_hbm, slab_v, lbl_i, pv, lv, sem, lsem):
    cid = lax.axis_index("c")
    sid = lax.axis_index("s")
    i = cid * _NS + sid
    active = i < _B
    iota = lax.iota(jnp.int32, 16)
    zeros = jnp.zeros((16,), jnp.float32)

    @pl.when(active)
    def _compute():
        lbl_cp = pltpu.async_copy(lbl_hbm, lbl_i, lsem)
        copies = [
            pltpu.async_copy(tp_hbm.at[i, pl.ds(off, ln)],
                             slab_v.at[pl.ds(off, ln)], sem)
            for off, ln in _CHUNKS
        ]
        lbl_cp.wait()
        ivec = jnp.full((16,), 0, jnp.int32) + i
        la = plsc.load_gather(lbl_i, [ivec, iota])
        lb = plsc.load_gather(lbl_i, [ivec, _clamp(iota + 16, 0, _L - 1)])
        l0 = _clamp(la, 0, _V - 1)
        l1 = _clamp(lb, 0, _V - 1)
        cntv = (jnp.where(la >= 0, 1.0, 0.0).astype(jnp.float32)
                + jnp.where((lb >= 0) & (iota < _L - 16),
                            1.0, 0.0).astype(jnp.float32))
        cnt = jnp.sum(cntv)

        def build_pv_rows(off, ln):
            for j in range(off, off + ln):
                vals0 = plsc.load_gather(slab_v, [jnp.full((16,), j, jnp.int32), l0])
                pv[pl.ds(j * 32, 16)] = vals0
                vals1 = plsc.load_gather(slab_v, [jnp.full((16,), j, jnp.int32), l1])
                pv[pl.ds(j * 32 + 16, 16)] = vals1

        def _shl(x):
            return x.at[_clamp(iota - 1, 0, 15)].get(mode="promise_in_bounds")

        def _lane(x, j):
            return x.at[jnp.full((16,), j, jnp.int32)].get(
                mode="promise_in_bounds")

        idxb = [(iota + 16 * h) * 31 - 33 for h in (0, 1)]
        a0 = a1 = b0 = b1 = zeros
        for sd in range(2, 2 * _L + 1):
            for m, (off, ln) in enumerate(_CHUNKS):
                if sd == off + 2:
                    copies[m].wait()
                    build_pv_rows(off, ln)
            rlo, rhi = max(1, sd - _L), min(_L, sd - 1)
            if rlo <= 15:
                r = iota
                valid = (r >= rlo) & (r <= rhi)
                p = plsc.load_gather(pv, [_clamp(idxb[0] + sd, 0, _L * 32 - 1)])
                mx = jnp.maximum(b0, _shl(b0))
                nv = p * (_shl(a0) + 1.0 - mx) + mx
                c0 = jnp.where(valid, nv, 0.0)
            else:
                c0 = a0
            if rhi >= 16:
                r = iota + 16
                valid = (r >= rlo) & (r <= rhi)
                p = plsc.load_gather(pv, [_clamp(idxb[1] + sd, 0, _L * 32 - 1)])
                lane0 = iota < 1
                am1 = jnp.where(lane0, _lane(a0, 15), _shl(a1))
                bm1 = jnp.where(lane0, _lane(b0, 15), _shl(b1))
                mx = jnp.maximum(b1, bm1)
                nv = p * (am1 + 1.0 - mx) + mx
                c1 = jnp.where(valid, nv, 0.0)
            else:
                c1 = a1
            a0, a1, b0, b1 = b0, b1, c0, c1
        dfin = _lane(b1, _L - 16)
        lnz = _ln16(dfin / cnt)
        lv[...] = lnz * jnp.float32(-1.0 / _B)

    @pl.when(jnp.logical_not(active))
    def _idle():
        lv[...] = zeros

    pltpu.sync_copy(lv, out_hbm.at[i, 0])


def kernel(topic_prob, hard_label):
    assert topic_prob.shape == (_B, _L, _V) and hard_label.shape == (_B, _L)
    out = _calcs_sc(topic_prob, hard_label)
    return jnp.sum(out[:, 0, 0])


# speedup vs baseline: 1.0012x; 1.0012x over previous
import functools

import jax
import jax.numpy as jnp
from jax import lax
from jax.experimental import pallas as pl
from jax.experimental.pallas import tpu as pltpu
from jax.experimental.pallas import tpu_sc as plsc

_B = 20
_L = 20
_V = 1000
_NS = 16
_NP = _L * _L
_CHUNKS = ((0, 8), (8, 8), (16, 4))
_LN2 = 0.6931471805599453


def _clamp(x, lo, hi):
    return jnp.minimum(jnp.maximum(x, lo), hi)


def _ln16(z):
    bits = plsc.bitcast(z, jnp.int32)
    e = lax.shift_right_logical(bits, 23) & 0xFF
    ef = (e - 127).astype(jnp.float32)
    m = plsc.bitcast((bits & 0x7FFFFF) | 0x3F800000, jnp.float32)
    t = (m - 1.0) / (m + 1.0)
    t2 = t * t
    p = jnp.float32(2.0 / 13.0)
    for coef in (2.0 / 11.0, 2.0 / 9.0, 2.0 / 7.0, 2.0 / 5.0, 2.0 / 3.0, 2.0):
        p = p * t2 + jnp.float32(coef)
    return ef * jnp.float32(_LN2) + t * p


@functools.partial(
    pl.kernel,
    out_type=jax.ShapeDtypeStruct((32, 1, 16), jnp.float32),
    mesh=plsc.VectorSubcoreMesh(core_axis_name="c", subcore_axis_name="s"),
    compiler_params=pltpu.CompilerParams(needs_layout_passes=False),
    scratch_types=[
        pltpu.VMEM((_L, _V), jnp.float32),
        pltpu.VMEM((_B, _L), jnp.int32),
        pltpu.VMEM((_L * 32,), jnp.float32),
        pltpu.VMEM((16,), jnp.float32),
        pltpu.SemaphoreType.DMA,
        pltpu.SemaphoreType.DMA,
    ],
)
def _calcs_sc(tp_hbm, lbl_hbm, out_hbm, slab_v, lbl_i, pv, lv, sem, lsem):
    cid = lax.axis_index("c")
    sid = lax.axis_index("s")
    i = cid * _NS + sid
    active = i < _B
    iota = lax.iota(jnp.int32, 16)
    zeros = jnp.zeros((16,), jnp.float32)

    @pl.when(active)
    def _compute():
        copies = [
            pltpu.async_copy(tp_hbm.at[i, pl.ds(off, ln)],
                             slab_v.at[pl.ds(off, ln)], sem)
            for off, ln in _CHUNKS
        ]
        lbl_cp = pltpu.async_copy(lbl_hbm, lbl_i, lsem)
        lbl_cp.wait()
        ivec = jnp.full((16,), 0, jnp.int32) + i
        la = plsc.load_gather(lbl_i, [ivec, iota])
        lb = plsc.load_gather(lbl_i, [ivec, _clamp(iota + 16, 0, _L - 1)])
        l0 = _clamp(la, 0, _V - 1)
        l1 = _clamp(lb, 0, _V - 1)
        cntv = (jnp.where(la >= 0, 1.0, 0.0).astype(jnp.float32)
                + jnp.where((lb >= 0) & (iota < _L - 16),
                            1.0, 0.0).astype(jnp.float32))
        cnt = jnp.sum(cntv)

        def build_pv_rows(off, ln):
            for j in range(off, off + ln):
                vals0 = plsc.load_gather(slab_v, [jnp.full((16,), j, jnp.int32), l0])
                pv[pl.ds(j * 32, 16)] = vals0
                vals1 = plsc.load_gather(slab_v, [jnp.full((16,), j, jnp.int32), l1])
                pv[pl.ds(j * 32 + 16, 16)] = vals1

        def _shl(x):
            return x.at[_clamp(iota - 1, 0, 15)].get(mode="promise_in_bounds")

        def _lane(x, j):
            return x.at[jnp.full((16,), j, jnp.int32)].get(
                mode="promise_in_bounds")

        idxb = [(iota + 16 * h) * 31 - 33 for h in (0, 1)]
        a0 = a1 = b0 = b1 = zeros
        for sd in range(2, 2 * _L + 1):
            for m, (off, ln) in enumerate(_CHUNKS):
                if sd == off + 2:
                    copies[m].wait()
                    build_pv_rows(off, ln)
            rlo, rhi = max(1, sd - _L), min(_L, sd - 1)
            if rlo <= 15:
                r = iota
                valid = (r >= rlo) & (r <= rhi)
                p = plsc.load_gather(pv, [_clamp(idxb[0] + sd, 0, _L * 32 - 1)])
                mx = jnp.maximum(b0, _shl(b0))
                nv = p * (_shl(a0) + 1.0 - mx) + mx
                c0 = jnp.where(valid, nv, 0.0)
            else:
                c0 = a0
            if rhi >= 16:
                r = iota + 16
                valid = (r >= rlo) & (r <= rhi)
                p = plsc.load_gather(pv, [_clamp(idxb[1] + sd, 0, _L * 32 - 1)])
                lane0 = iota < 1
                am1 = jnp.where(lane0, _lane(a0, 15), _shl(a1))
                bm1 = jnp.where(lane0, _lane(b0, 15), _shl(b1))
                mx = jnp.maximum(b1, bm1)
                nv = p * (am1 + 1.0 - mx) + mx
                c1 = jnp.where(valid, nv, 0.0)
            else:
                c1 = a1
            a0, a1, b0, b1 = b0, b1, c0, c1
        dfin = _lane(b1, _L - 16)
        lnz = _ln16(dfin / cnt)
        lv[...] = lnz * jnp.float32(-1.0 / _B)

    @pl.when(jnp.logical_not(active))
    def _idle():
        lv[...] = zeros

    pltpu.sync_copy(lv, out_hbm.at[i, 0])


def kernel(topic_prob, hard_label):
    assert topic_prob.shape == (_B, _L, _V) and hard_label.shape == (_B, _L)
    out = _calcs_sc(topic_prob, hard_label)
    return jnp.sum(out[:, 0, 0])


# speedup vs baseline: 1.0088x; 1.0077x over previous
import functools

import jax
import jax.numpy as jnp
from jax import lax
from jax.experimental import pallas as pl
from jax.experimental.pallas import tpu as pltpu
from jax.experimental.pallas import tpu_sc as plsc

_B = 20
_L = 20
_V = 1000
_NS = 16
_NP = _L * _L
_CHUNKS = ((0, 8), (8, 8), (16, 4))
_LN2 = 0.6931471805599453


def _clamp(x, lo, hi):
    return jnp.minimum(jnp.maximum(x, lo), hi)


def _ln16(z):
    bits = plsc.bitcast(z, jnp.int32)
    e = lax.shift_right_logical(bits, 23) & 0xFF
    ef = (e - 127).astype(jnp.float32)
    m = plsc.bitcast((bits & 0x7FFFFF) | 0x3F800000, jnp.float32)
    t = (m - 1.0) / (m + 1.0)
    t2 = t * t
    p = jnp.float32(2.0 / 13.0)
    for coef in (2.0 / 11.0, 2.0 / 9.0, 2.0 / 7.0, 2.0 / 5.0, 2.0 / 3.0, 2.0):
        p = p * t2 + jnp.float32(coef)
    return ef * jnp.float32(_LN2) + t * p


@functools.partial(
    pl.kernel,
    out_type=jax.ShapeDtypeStruct((32, 1, 16), jnp.float32),
    mesh=plsc.VectorSubcoreMesh(core_axis_name="c", subcore_axis_name="s"),
    compiler_params=pltpu.CompilerParams(needs_layout_passes=False),
    scratch_types=[
        pltpu.VMEM((_L, _V), jnp.float32),
        pltpu.VMEM((_B, _L), jnp.int32),
        pltpu.VMEM((_NP,), jnp.float32),
        pltpu.VMEM((16,), jnp.float32),
        pltpu.SemaphoreType.DMA,
        pltpu.SemaphoreType.DMA,
    ],
)
def _calcs_sc(tp_hbm, lbl_hbm, out_hbm, slab_v, lbl_i, pv, lv, sem, lsem):
    cid = lax.axis_index("c")
    sid = lax.axis_index("s")
    i = cid * _NS + sid
    active = i < _B
    iota = lax.iota(jnp.int32, 16)
    zeros = jnp.zeros((16,), jnp.float32)

    @pl.when(active)
    def _compute():
        copies = [
            pltpu.async_copy(tp_hbm.at[i, pl.ds(off, ln)],
                             slab_v.at[pl.ds(off, ln)], sem)
            for off, ln in _CHUNKS
        ]
        lbl_cp = pltpu.async_copy(lbl_hbm, lbl_i, lsem)
        lbl_cp.wait()
        ivec = jnp.full((16,), 0, jnp.int32) + i
        la = plsc.load_gather(lbl_i, [ivec, iota])
        lb = plsc.load_gather(lbl_i, [ivec, _clamp(iota + 16, 0, _L - 1)])
        l0 = _clamp(la, 0, _V - 1)
        l1 = _clamp(lb, 0, _V - 1)
        cntv = (jnp.where(la >= 0, 1.0, 0.0).astype(jnp.float32)
                + jnp.where((lb >= 0) & (iota < _L - 16),
                            1.0, 0.0).astype(jnp.float32))
        cnt = jnp.sum(cntv)

        def build_pv_rows(off, ln):
            for j in range(off, off + ln):
                vals0 = plsc.load_gather(slab_v, [jnp.full((16,), j, jnp.int32), l0])
                plsc.store_scatter(pv, [iota + j * _L], vals0)
                vals1 = plsc.load_gather(slab_v, [jnp.full((16,), j, jnp.int32), l1])
                plsc.store_scatter(pv, [iota + (j * _L + 16)], vals1,
                                   mask=iota < (_L - 16))

        def _shl(x):
            return x.at[_clamp(iota - 1, 0, 15)].get(mode="promise_in_bounds")

        def _lane(x, j):
            return x.at[jnp.full((16,), j, jnp.int32)].get(
                mode="promise_in_bounds")

        idxb = [(iota + 16 * h) * (_L - 1) - (_L + 1) for h in (0, 1)]
        a0 = a1 = b0 = b1 = zeros
        for sd in range(2, 2 * _L + 1):
            for m, (off, ln) in enumerate(_CHUNKS):
                if sd == off + 2:
                    copies[m].wait()
                    build_pv_rows(off, ln)
            rlo, rhi = max(1, sd - _L), min(_L, sd - 1)
            if rlo <= 15:
                r = iota
                valid = (r >= rlo) & (r <= rhi)
                p = plsc.load_gather(pv, [_clamp(idxb[0] + sd, 0, _NP - 1)])
                mx = jnp.maximum(b0, _shl(b0))
                nv = p * (_shl(a0) + 1.0 - mx) + mx
                c0 = jnp.where(valid, nv, 0.0)
            else:
                c0 = a0
            if rhi >= 16:
                r = iota + 16
                valid = (r >= rlo) & (r <= rhi)
                p = plsc.load_gather(pv, [_clamp(idxb[1] + sd, 0, _NP - 1)])
                lane0 = iota < 1
                am1 = jnp.where(lane0, _lane(a0, 15), _shl(a1))
                bm1 = jnp.where(lane0, _lane(b0, 15), _shl(b1))
                mx = jnp.maximum(b1, bm1)
                nv = p * (am1 + 1.0 - mx) + mx
                c1 = jnp.where(valid, nv, 0.0)
            else:
                c1 = a1
            a0, a1, b0, b1 = b0, b1, c0, c1
        dfin = _lane(b1, _L - 16)
        lnz = _ln16(dfin / cnt)
        lv[...] = lnz * jnp.float32(-1.0 / _B)

    @pl.when(jnp.logical_not(active))
    def _idle():
        lv[...] = zeros

    pltpu.sync_copy(lv, out_hbm.at[i, 0])


def kernel(topic_prob, hard_label):
    assert topic_prob.shape == (_B, _L, _V) and hard_label.shape == (_B, _L)
    out = _calcs_sc(topic_prob, hard_label)
    return jnp.sum(out[:, 0, 0])
